# baseline (speedup 1.0000x reference)
import jax
import jax.numpy as jnp
from jax.experimental import pallas as pl


N_NODES = 10000
N_MSG = 32
N_FEAT = 128
N_BLK = 400


def _reduce_body(x_ref, o_ref):
    x = x_ref[...].reshape(N_BLK, N_MSG, N_FEAT)
    o_ref[...] = jnp.sum(x, axis=1)


def kernel(messages):
    x = messages.reshape(N_NODES * N_MSG, N_FEAT)
    out = pl.pallas_call(
        _reduce_body,
        grid=(N_NODES // N_BLK,),
        in_specs=[pl.BlockSpec((N_BLK * N_MSG, N_FEAT), lambda i: (i, 0))],
        out_specs=pl.BlockSpec((N_BLK, N_FEAT), lambda i: (i, 0)),
        out_shape=jax.ShapeDtypeStruct((N_NODES, N_FEAT), jnp.float32),
    )(x)
    return out.reshape(1, N_NODES, N_FEAT)


# speedup vs baseline: 1.0076x; 1.0076x over previous
import jax
import jax.numpy as jnp
from jax.experimental import pallas as pl


N_NODES = 10000
N_MSG = 32
N_FEAT = 128
N_BLK = 400


def _reduce_body(x_ref, o_ref):
    o_ref[...] = jnp.sum(x_ref[...], axis=1)


def kernel(messages):
    x = messages.reshape(N_NODES, N_MSG, N_FEAT)
    out = pl.pallas_call(
        _reduce_body,
        grid=(N_NODES // N_BLK,),
        in_specs=[pl.BlockSpec((N_BLK, N_MSG, N_FEAT), lambda i: (i, 0, 0))],
        out_specs=pl.BlockSpec((N_BLK, N_FEAT), lambda i: (i, 0)),
        out_shape=jax.ShapeDtypeStruct((N_NODES, N_FEAT), jnp.float32),
    )(x)
    return out.reshape(1, N_NODES, N_FEAT)
